# Initial kernel scaffold; baseline (speedup 1.0000x reference)
import jax
import jax.numpy as jnp
from jax.experimental import pallas as pl


def kernel(X, edge_idx, C):
    raise NotImplementedError("write your pallas kernel here")


# speedup vs baseline: 9.6466x; 9.6466x over previous
import functools

import jax
import jax.numpy as jnp
from jax import lax
from jax.experimental import pallas as pl
from jax.experimental.pallas import tpu as pltpu
from jax.experimental.pallas import tpu_sc as plsc

N = 50000
K = 32
E_TOT = N * K
NC, NS = 2, 16
NW = NC * NS
PER_W = E_TOT // NW
E_C = 10000
CHUNKS = PER_W // E_C
VEC = 16
LN2 = 0.6931471805599453
SQRT2 = 1.4142135623730951


def _ln1p_abs(d):
    x = jnp.abs(d) + 1.0
    bits = plsc.bitcast(x, jnp.int32)
    e = (bits >> 23) - 127
    m = plsc.bitcast((bits & 0x007FFFFF) | 0x3F800000, jnp.float32)
    big = m > SQRT2
    m = jnp.where(big, m * 0.5, m)
    e_f = (e + big.astype(jnp.int32)).astype(jnp.float32)
    s = (m - 1.0) / (m + 1.0)
    s2 = s * s
    p = 2.0 * s * (1.0 + s2 * (1.0 / 3.0 + s2 * (1.0 / 5.0 + s2 * (1.0 / 7.0))))
    return e_f * LN2 + p


def _sc_body(edge_hbm, c_hbm, out_hbm, c_v, idx_v, out_v):
    wid = lax.axis_index("s") * NC + lax.axis_index("c")
    pltpu.sync_copy(c_hbm, c_v)
    base_e = wid * PER_W

    def chunk_body(ch, _):
        ebase = base_e + ch * E_C
        pltpu.sync_copy(edge_hbm.at[pl.ds(ebase, E_C)], idx_v)

        def vec_body(v, _):
            off = v * VEC
            lane = lax.iota(jnp.int32, VEC)
            j = idx_v[pl.ds(off, VEC)]
            i = (ebase + off + lane) >> 5
            ci = plsc.load_gather(c_v, [i])
            cj = plsc.load_gather(c_v, [j])
            same = ci == cj
            d = (j - i).astype(jnp.float32)
            isif = jnp.where(same, 0.0, 1.0)
            intra = jnp.where(same, _ln1p_abs(d), 0.0)
            sgn = jnp.where(same, jnp.sign(d), 0.0)
            p0 = 3 * (off + lane)
            plsc.store_scatter(out_v, [p0], isif)
            plsc.store_scatter(out_v, [p0 + 1], intra)
            plsc.store_scatter(out_v, [p0 + 2], sgn)
            return 0

        lax.fori_loop(0, E_C // VEC, vec_body, 0)
        pltpu.sync_copy(out_v, out_hbm.at[pl.ds(3 * ebase, 3 * E_C)])
        return 0

    lax.fori_loop(0, CHUNKS, chunk_body, 0)


@jax.jit
def _edge_field(edge_flat, c_flat):
    mesh = plsc.VectorSubcoreMesh(
        core_axis_name="c", subcore_axis_name="s",
        num_cores=NC, num_subcores=NS)
    run = pl.kernel(
        _sc_body,
        out_type=jax.ShapeDtypeStruct((E_TOT * 3,), jnp.float32),
        mesh=mesh,
        scratch_types=[
            pltpu.VMEM((N,), jnp.int32),
            pltpu.VMEM((E_C,), jnp.int32),
            pltpu.VMEM((3 * E_C,), jnp.float32),
        ],
        compiler_params=pltpu.CompilerParams(needs_layout_passes=False),
    )
    return run(edge_flat, c_flat)


def kernel(X, edge_idx, C):
    del X
    edge_flat = edge_idx.reshape(E_TOT)
    c_flat = C.reshape(N)
    out = _edge_field(edge_flat, c_flat)
    return out.reshape(1, N, K, 3)


# speedup vs baseline: 10.3372x; 1.0716x over previous
import jax
import jax.numpy as jnp
from jax import lax
from jax.experimental import pallas as pl
from jax.experimental.pallas import tpu as pltpu
from jax.experimental.pallas import tpu_sc as plsc

N = 50000
K = 32
E_TOT = N * K
NC, NS = 2, 16
NW = NC * NS
PER_W = E_TOT // NW
E_C = 10000
CHUNKS = PER_W // E_C
VEC = 16
LN2 = 0.6931471805599453
P0 = -2.4967737679054736
P1 = 4.028372766846634
P2 = -2.081060203459175
P3 = 0.6288157291848091
P4 = -0.07915036575315018


def _ln1p_abs(d):
    x = jnp.abs(d) + 1.0
    bits = plsc.bitcast(x, jnp.int32)
    e = ((bits >> 23) - 127).astype(jnp.float32)
    m = plsc.bitcast((bits & 0x007FFFFF) | 0x3F800000, jnp.float32)
    p = ((((P4 * m + P3) * m + P2) * m + P1) * m) + P0
    return (e + p) * LN2


def _sc_body(edge_hbm, c_hbm, out_hbm,
             c_v, idx0, idx1, out0, out1,
             sem_c, si0, si1, so0, so1):
    wid = lax.axis_index("s") * NC + lax.axis_index("c")
    base_e = wid * PER_W
    idx_bufs, out_bufs = (idx0, idx1), (out0, out1)
    si, so = (si0, si1), (so0, so1)

    c_cp = pltpu.async_copy(c_hbm, c_v, sem_c)
    in_cps = [pltpu.async_copy(edge_hbm.at[pl.ds(base_e, E_C)], idx0, si0),
              None]
    out_cps = [None, None]
    c_cp.wait()

    for ch in range(CHUNKS):
        cur = ch & 1
        nxt = 1 - cur
        if ch + 1 < CHUNKS:
            in_cps[nxt] = pltpu.async_copy(
                edge_hbm.at[pl.ds(base_e + (ch + 1) * E_C, E_C)],
                idx_bufs[nxt], si[nxt])
        in_cps[cur].wait()
        if out_cps[cur] is not None:
            out_cps[cur].wait()

        ebase = base_e + ch * E_C
        idx_v, out_v = idx_bufs[cur], out_bufs[cur]

        @plsc.parallel_loop(0, E_C, step=VEC, unroll=5)
        def _vec(off):
            lane = lax.iota(jnp.int32, VEC)
            j = idx_v[pl.ds(off, VEC)]
            i = (ebase + off + lane) >> 5
            ci = plsc.load_gather(c_v, [i])
            cj = plsc.load_gather(c_v, [j])
            same = jnp.where(ci == cj, 1.0, 0.0)
            d = (j - i).astype(jnp.float32)
            isif = 1.0 - same
            intra = same * _ln1p_abs(d)
            sgn = same * jnp.sign(d)
            p0 = 3 * off + lane * 3
            plsc.store_scatter(out_v, [p0], isif)
            plsc.store_scatter(out_v, [p0 + 1], intra)
            plsc.store_scatter(out_v, [p0 + 2], sgn)

        out_cps[cur] = pltpu.async_copy(
            out_v, out_hbm.at[pl.ds(3 * ebase, 3 * E_C)], so[cur])

    for cp in out_cps:
        if cp is not None:
            cp.wait()


@jax.jit
def _edge_field(edge_flat, c_flat):
    mesh = plsc.VectorSubcoreMesh(
        core_axis_name="c", subcore_axis_name="s",
        num_cores=NC, num_subcores=NS)
    run = pl.kernel(
        _sc_body,
        out_type=jax.ShapeDtypeStruct((E_TOT * 3,), jnp.float32),
        mesh=mesh,
        scratch_types=[
            pltpu.VMEM((N,), jnp.int32),
            pltpu.VMEM((E_C,), jnp.int32),
            pltpu.VMEM((E_C,), jnp.int32),
            pltpu.VMEM((3 * E_C,), jnp.float32),
            pltpu.VMEM((3 * E_C,), jnp.float32),
            pltpu.SemaphoreType.DMA,
            pltpu.SemaphoreType.DMA,
            pltpu.SemaphoreType.DMA,
            pltpu.SemaphoreType.DMA,
            pltpu.SemaphoreType.DMA,
        ],
        compiler_params=pltpu.CompilerParams(needs_layout_passes=False),
    )
    return run(edge_flat, c_flat)


def kernel(X, edge_idx, C):
    del X
    edge_flat = edge_idx.reshape(E_TOT)
    c_flat = C.reshape(N)
    out = _edge_field(edge_flat, c_flat)
    return out.reshape(1, N, K, 3)


# speedup vs baseline: 197.2797x; 19.0845x over previous
import jax
import jax.numpy as jnp
from jax import lax
from jax.experimental import pallas as pl
from jax.experimental.pallas import tpu as pltpu
from jax.experimental.pallas import tpu_sc as plsc

N = 50000
K = 32
NC, NS = 2, 16
NW = NC * NS
KB = 8
NODES_W = 6144
E_C = 768
CHUNKS = NODES_W // E_C
MAIN_N = 8 * NODES_W
FULL_T0 = MAIN_N // 128
N_FULL_T = 6
TAIL_N = FULL_T0 * 128 + N_FULL_T * 128
TAIL_LEN = N - TAIL_N
VEC = 16
LN2 = 0.6931471805599453
P0 = -2.4967737679054736
P1 = 4.028372766846634
P2 = -2.081060203459175
P3 = 0.6288157291848091
P4 = -0.07915036575315018


def _ln1p_abs(d):
    x = jnp.abs(d) + 1.0
    bits = plsc.bitcast(x, jnp.int32)
    e = ((bits >> 23) - 127).astype(jnp.float32)
    m = plsc.bitcast((bits & 0x007FFFFF) | 0x3F800000, jnp.float32)
    p = ((((P4 * m + P3) * m + P2) * m + P1) * m) + P0
    return (e + p) * LN2


def _cols16(c_v, idx_v, out_v, off, nb, clamp=False):
    i = nb + off + lax.iota(jnp.int32, VEC)
    ci = c_v[pl.ds(nb + off, VEC)]
    for r in range(KB):
        j = idx_v[r, pl.ds(off, VEC)]
        if clamp:
            j = jnp.clip(j, 0, N - 1)
        cj = plsc.load_gather(c_v, [j])
        same = jnp.where(ci == cj, 1.0, 0.0)
        d = (j - i).astype(jnp.float32)
        out_v[0, r, pl.ds(off, VEC)] = 1.0 - same
        out_v[1, r, pl.ds(off, VEC)] = same * _ln1p_abs(d)
        out_v[2, r, pl.ds(off, VEC)] = same * jnp.sign(d)


def _sc_body(edge_hbm, c_hbm, out_hbm,
             c_v, idx0, idx1, out0, out1,
             sem_c, si0, si1, so0, so1):
    wid = lax.axis_index("s") * NC + lax.axis_index("c")
    kb8 = pl.multiple_of((wid & 3) * KB, KB)
    base_n = (wid >> 2) * NODES_W
    idx_bufs, out_bufs = (idx0, idx1), (out0, out1)
    si, so = (si0, si1), (so0, so1)

    def in_slice(nb, cols=E_C):
        return edge_hbm.at[pl.ds(kb8, KB), pl.ds(pl.multiple_of(nb, 128), cols)]

    def out_slice(nb, cols=E_C):
        return out_hbm.at[:, pl.ds(kb8, KB), pl.ds(pl.multiple_of(nb, 128), cols)]

    c_cp = pltpu.async_copy(c_hbm, c_v.at[pl.ds(0, N)], sem_c)
    pltpu.async_copy(in_slice(base_n), idx0, si0)
    pltpu.async_copy(in_slice(base_n + E_C), idx1, si1)
    c_cp.wait()

    def ring(g, _):
        for b in (0, 1):
            ch = 2 * g + b
            nb = base_n + ch * E_C
            idx_v, out_v = idx_bufs[b], out_bufs[b]
            pltpu.make_async_copy(in_slice(nb), idx_v, si[b]).wait()

            @pl.when(g > 0)
            def _():
                pltpu.make_async_copy(out_v, out_slice(nb), so[b]).wait()

            @plsc.parallel_loop(0, E_C, step=VEC, unroll=2)
            def _col(off):
                _cols16(c_v, idx_v, out_v, off, nb)

            pltpu.async_copy(out_v, out_slice(nb), so[b])

            @pl.when(ch + 2 < CHUNKS)
            def _():
                pltpu.async_copy(
                    in_slice(base_n + (ch + 2) * E_C), idx_v, si[b])
        return 0

    lax.fori_loop(0, CHUNKS // 2, ring, 0)
    for b in (0, 1):
        nb = base_n + (CHUNKS - 2 + b) * E_C
        pltpu.make_async_copy(out_bufs[b], out_slice(nb), so[b]).wait()

    @pl.when(wid < 4 * N_FULL_T)
    def _extra():
        nb = (FULL_T0 + (wid >> 2)) * 128
        pltpu.sync_copy(in_slice(nb, 128), idx0.at[:, pl.ds(0, 128)])

        @plsc.parallel_loop(0, 128, step=VEC, unroll=2)
        def _col(off):
            _cols16(c_v, idx0, out0, off, nb)

        pltpu.sync_copy(out0.at[:, :, pl.ds(0, 128)], out_slice(nb, 128))

    @pl.when((wid >= 28) & (wid < 32))
    def _tail():
        kb8_t = pl.multiple_of((wid - 28) * KB, KB)
        in_ref = edge_hbm.at[pl.ds(kb8_t, KB),
                             pl.ds(pl.multiple_of(TAIL_N, 128), 128)]
        pltpu.sync_copy(in_ref, idx0.at[:, pl.ds(0, 128)])

        @plsc.parallel_loop(0, 128, step=VEC, unroll=2)
        def _col(off):
            _cols16(c_v, idx0, out0, off, TAIL_N, clamp=True)

        out_ref = out_hbm.at[:, pl.ds(kb8_t, KB),
                             pl.ds(pl.multiple_of(TAIL_N, 128), 128)]
        pltpu.sync_copy(out0.at[:, :, pl.ds(0, 128)], out_ref)


@jax.jit
def _edge_field(edge_t, c_flat):
    mesh = plsc.VectorSubcoreMesh(
        core_axis_name="c", subcore_axis_name="s",
        num_cores=NC, num_subcores=NS)
    run = pl.kernel(
        _sc_body,
        out_type=jax.ShapeDtypeStruct((3, K, N), jnp.float32),
        mesh=mesh,
        scratch_types=[
            pltpu.VMEM((N + 48,), jnp.int32),
            pltpu.VMEM((KB, E_C), jnp.int32),
            pltpu.VMEM((KB, E_C), jnp.int32),
            pltpu.VMEM((3, KB, E_C), jnp.float32),
            pltpu.VMEM((3, KB, E_C), jnp.float32),
            pltpu.SemaphoreType.DMA,
            pltpu.SemaphoreType.DMA,
            pltpu.SemaphoreType.DMA,
            pltpu.SemaphoreType.DMA,
            pltpu.SemaphoreType.DMA,
        ],
        compiler_params=pltpu.CompilerParams(needs_layout_passes=False),
    )
    return run(edge_t, c_flat)


def kernel(X, edge_idx, C):
    del X
    edge_t = jnp.transpose(edge_idx[0], (1, 0))
    out_t = _edge_field(edge_t, C.reshape(N))
    return jnp.transpose(out_t, (2, 1, 0))[None]


# speedup vs baseline: 216.0358x; 1.0951x over previous
import jax
import jax.numpy as jnp
from jax import lax
from jax.experimental import pallas as pl
from jax.experimental.pallas import tpu as pltpu
from jax.experimental.pallas import tpu_sc as plsc

N = 50000
K = 32
NC, NS = 2, 16
NW = NC * NS
KB = 8
NODES_W = 6144
E_C = 768
CHUNKS = NODES_W // E_C
MAIN_N = 8 * NODES_W
FULL_T0 = MAIN_N // 128
N_FULL_T = 6
TAIL_N = FULL_T0 * 128 + N_FULL_T * 128
TAIL_LEN = N - TAIL_N
VEC = 16
LN2 = 0.6931471805599453
Q0 = (-2.4967737679054736 - 127.0) * LN2
Q1 = 4.028372766846634 * LN2
Q2 = -2.081060203459175 * LN2
Q3 = 0.6288157291848091 * LN2
Q4 = -0.07915036575315018 * LN2


def _ln1p_abs(d):
    x = jnp.abs(d) + 1.0
    bits = plsc.bitcast(x, jnp.int32)
    e = (bits >> 23).astype(jnp.float32)
    m = plsc.bitcast((bits & 0x007FFFFF) | 0x3F800000, jnp.float32)
    p = ((((Q4 * m + Q3) * m + Q2) * m + Q1) * m) + Q0
    return e * LN2 + p


def _cols16(c_v, idx_v, out_v, off, nb, clamp=False):
    i = nb + off + lax.iota(jnp.int32, VEC)
    ci = c_v[pl.ds(nb + off, VEC)]
    for r in range(KB):
        j = idx_v[r, pl.ds(off, VEC)]
        if clamp:
            j = jnp.clip(j, 0, N - 1)
        cj = plsc.load_gather(c_v, [j])
        eq = ci == cj
        d = (j - i).astype(jnp.float32)
        out_v[0, r, pl.ds(off, VEC)] = jnp.where(eq, 0.0, 1.0)
        out_v[1, r, pl.ds(off, VEC)] = jnp.where(eq, _ln1p_abs(d), 0.0)
        out_v[2, r, pl.ds(off, VEC)] = jnp.where(eq, jnp.sign(d), 0.0)


def _sc_body(edge_hbm, c_hbm, out_hbm,
             c_v, idx0, idx1, out0, out1,
             sem_c, si0, si1, so0, so1):
    wid = lax.axis_index("s") * NC + lax.axis_index("c")
    kb8 = pl.multiple_of((wid & 3) * KB, KB)
    base_n = (wid >> 2) * NODES_W
    idx_bufs, out_bufs = (idx0, idx1), (out0, out1)
    si, so = (si0, si1), (so0, so1)

    def in_slice(nb, cols=E_C):
        return edge_hbm.at[pl.ds(kb8, KB), pl.ds(pl.multiple_of(nb, 128), cols)]

    def out_slice(nb, cols=E_C):
        return out_hbm.at[:, pl.ds(kb8, KB), pl.ds(pl.multiple_of(nb, 128), cols)]

    c_cp = pltpu.async_copy(c_hbm, c_v.at[pl.ds(0, N)], sem_c)
    pltpu.async_copy(in_slice(base_n), idx0, si0)
    pltpu.async_copy(in_slice(base_n + E_C), idx1, si1)
    c_cp.wait()

    def ring(g, _):
        for b in (0, 1):
            ch = 2 * g + b
            nb = base_n + ch * E_C
            idx_v, out_v = idx_bufs[b], out_bufs[b]
            pltpu.make_async_copy(in_slice(nb), idx_v, si[b]).wait()

            @pl.when(g > 0)
            def _():
                pltpu.make_async_copy(out_v, out_slice(nb), so[b]).wait()

            @plsc.parallel_loop(0, E_C, step=VEC, unroll=2)
            def _col(off):
                _cols16(c_v, idx_v, out_v, off, nb)

            pltpu.async_copy(out_v, out_slice(nb), so[b])

            @pl.when(ch + 2 < CHUNKS)
            def _():
                pltpu.async_copy(
                    in_slice(base_n + (ch + 2) * E_C), idx_v, si[b])
        return 0

    lax.fori_loop(0, CHUNKS // 2, ring, 0)
    for b in (0, 1):
        nb = base_n + (CHUNKS - 2 + b) * E_C
        pltpu.make_async_copy(out_bufs[b], out_slice(nb), so[b]).wait()

    @pl.when(wid < 4 * N_FULL_T)
    def _extra():
        nb = (FULL_T0 + (wid >> 2)) * 128
        pltpu.sync_copy(in_slice(nb, 128), idx0.at[:, pl.ds(0, 128)])

        @plsc.parallel_loop(0, 128, step=VEC, unroll=2)
        def _col(off):
            _cols16(c_v, idx0, out0, off, nb)

        pltpu.sync_copy(out0.at[:, :, pl.ds(0, 128)], out_slice(nb, 128))

    @pl.when((wid >= 28) & (wid < 32))
    def _tail():
        kb8_t = pl.multiple_of((wid - 28) * KB, KB)
        in_ref = edge_hbm.at[pl.ds(kb8_t, KB),
                             pl.ds(pl.multiple_of(TAIL_N, 128), 128)]
        pltpu.sync_copy(in_ref, idx0.at[:, pl.ds(0, 128)])

        @plsc.parallel_loop(0, 128, step=VEC, unroll=2)
        def _col(off):
            _cols16(c_v, idx0, out0, off, TAIL_N, clamp=True)

        out_ref = out_hbm.at[:, pl.ds(kb8_t, KB),
                             pl.ds(pl.multiple_of(TAIL_N, 128), 128)]
        pltpu.sync_copy(out0.at[:, :, pl.ds(0, 128)], out_ref)


@jax.jit
def _edge_field(edge_t, c_flat):
    mesh = plsc.VectorSubcoreMesh(
        core_axis_name="c", subcore_axis_name="s",
        num_cores=NC, num_subcores=NS)
    run = pl.kernel(
        _sc_body,
        out_type=jax.ShapeDtypeStruct((3, K, N), jnp.float32),
        mesh=mesh,
        scratch_types=[
            pltpu.VMEM((N + 48,), jnp.int32),
            pltpu.VMEM((KB, E_C), jnp.int32),
            pltpu.VMEM((KB, E_C), jnp.int32),
            pltpu.VMEM((3, KB, E_C), jnp.float32),
            pltpu.VMEM((3, KB, E_C), jnp.float32),
            pltpu.SemaphoreType.DMA,
            pltpu.SemaphoreType.DMA,
            pltpu.SemaphoreType.DMA,
            pltpu.SemaphoreType.DMA,
            pltpu.SemaphoreType.DMA,
        ],
        compiler_params=pltpu.CompilerParams(needs_layout_passes=False),
    )
    return run(edge_t, c_flat)


def kernel(X, edge_idx, C):
    del X
    edge_t = jnp.transpose(edge_idx[0], (1, 0))
    out_t = _edge_field(edge_t, C.reshape(N))
    return jnp.transpose(out_t, (2, 1, 0))[None]


# speedup vs baseline: 245.9836x; 1.1386x over previous
import jax
import jax.numpy as jnp
from jax import lax
from jax.experimental import pallas as pl
from jax.experimental.pallas import tpu as pltpu
from jax.experimental.pallas import tpu_sc as plsc

N = 50000
K = 32
NC, NS = 2, 16
NW = NC * NS
KB = 8
NODES_W = 6144
E_C = 384
CHUNKS = NODES_W // E_C
MAIN_N = 8 * NODES_W
FULL_T0 = MAIN_N // 128
N_FULL_T = 6
TAIL_N = FULL_T0 * 128 + N_FULL_T * 128
TAIL_LEN = N - TAIL_N
VEC = 16


def _cols16(c_v, lt_v, idx_v, out_v, off, nb, clamp=False):
    i = nb + off + lax.iota(jnp.int32, VEC)
    ci = c_v[pl.ds(nb + off, VEC)]
    for r in range(KB):
        j = idx_v[r, pl.ds(off, VEC)]
        if clamp:
            j = jnp.clip(j, 0, N - 1)
        cj = plsc.load_gather(c_v, [j])
        eq = ci == cj
        di = j - i
        ad = jnp.abs(di)
        lnv = plsc.load_gather(lt_v, [ad])
        sgn = jnp.sign(di.astype(jnp.float32))
        out_v[0, r, pl.ds(off, VEC)] = jnp.where(eq, 0.0, 1.0)
        out_v[1, r, pl.ds(off, VEC)] = jnp.where(eq, lnv, 0.0)
        out_v[2, r, pl.ds(off, VEC)] = jnp.where(eq, sgn, 0.0)


def _sc_body(edge_hbm, c_hbm, lt_hbm, out_hbm,
             c_v, lt_v, idx0, idx1, out0, out1,
             sem_c, sem_lt, si0, si1, so0, so1):
    wid = lax.axis_index("s") * NC + lax.axis_index("c")
    kb8 = pl.multiple_of((wid & 3) * KB, KB)
    base_n = (wid >> 2) * NODES_W
    idx_bufs, out_bufs = (idx0, idx1), (out0, out1)
    si, so = (si0, si1), (so0, so1)

    def in_slice(nb, cols=E_C):
        return edge_hbm.at[pl.ds(kb8, KB), pl.ds(pl.multiple_of(nb, 128), cols)]

    def out_slice(nb, cols=E_C):
        return out_hbm.at[:, pl.ds(kb8, KB), pl.ds(pl.multiple_of(nb, 128), cols)]

    c_cp = pltpu.async_copy(c_hbm, c_v.at[pl.ds(0, N)], sem_c)
    lt_cp = pltpu.async_copy(lt_hbm, lt_v.at[pl.ds(0, N)], sem_lt)
    pltpu.async_copy(in_slice(base_n), idx0, si0)
    pltpu.async_copy(in_slice(base_n + E_C), idx1, si1)
    c_cp.wait()
    lt_cp.wait()

    def ring(g, _):
        for b in (0, 1):
            ch = 2 * g + b
            nb = base_n + ch * E_C
            idx_v, out_v = idx_bufs[b], out_bufs[b]
            pltpu.make_async_copy(in_slice(nb), idx_v, si[b]).wait()

            @pl.when(g > 0)
            def _():
                pltpu.make_async_copy(out_v, out_slice(nb), so[b]).wait()

            @plsc.parallel_loop(0, E_C, step=VEC, unroll=2)
            def _col(off):
                _cols16(c_v, lt_v, idx_v, out_v, off, nb)

            pltpu.async_copy(out_v, out_slice(nb), so[b])

            @pl.when(ch + 2 < CHUNKS)
            def _():
                pltpu.async_copy(
                    in_slice(base_n + (ch + 2) * E_C), idx_v, si[b])
        return 0

    lax.fori_loop(0, CHUNKS // 2, ring, 0)
    for b in (0, 1):
        nb = base_n + (CHUNKS - 2 + b) * E_C
        pltpu.make_async_copy(out_bufs[b], out_slice(nb), so[b]).wait()

    @pl.when(wid < 4 * N_FULL_T)
    def _extra():
        nb = (FULL_T0 + (wid >> 2)) * 128
        pltpu.sync_copy(in_slice(nb, 128), idx0.at[:, pl.ds(0, 128)])

        @plsc.parallel_loop(0, 128, step=VEC, unroll=2)
        def _col(off):
            _cols16(c_v, lt_v, idx0, out0, off, nb)

        pltpu.sync_copy(out0.at[:, :, pl.ds(0, 128)], out_slice(nb, 128))

    @pl.when((wid >= 28) & (wid < 32))
    def _tail():
        kb8_t = pl.multiple_of((wid - 28) * KB, KB)
        in_ref = edge_hbm.at[pl.ds(kb8_t, KB),
                             pl.ds(pl.multiple_of(TAIL_N, 128), 128)]
        pltpu.sync_copy(in_ref, idx0.at[:, pl.ds(0, 128)])

        @plsc.parallel_loop(0, 128, step=VEC, unroll=2)
        def _col(off):
            _cols16(c_v, lt_v, idx0, out0, off, TAIL_N, clamp=True)

        out_ref = out_hbm.at[:, pl.ds(kb8_t, KB),
                             pl.ds(pl.multiple_of(TAIL_N, 128), 128)]
        pltpu.sync_copy(out0.at[:, :, pl.ds(0, 128)], out_ref)


@jax.jit
def _edge_field(edge_t, c_flat):
    log_tab = jnp.log1p(jnp.arange(N, dtype=jnp.float32))
    mesh = plsc.VectorSubcoreMesh(
        core_axis_name="c", subcore_axis_name="s",
        num_cores=NC, num_subcores=NS)
    run = pl.kernel(
        _sc_body,
        out_type=jax.ShapeDtypeStruct((3, K, N), jnp.float32),
        mesh=mesh,
        scratch_types=[
            pltpu.VMEM((N + 48,), jnp.int32),
            pltpu.VMEM((N + 48,), jnp.float32),
            pltpu.VMEM((KB, E_C), jnp.int32),
            pltpu.VMEM((KB, E_C), jnp.int32),
            pltpu.VMEM((3, KB, E_C), jnp.float32),
            pltpu.VMEM((3, KB, E_C), jnp.float32),
            pltpu.SemaphoreType.DMA,
            pltpu.SemaphoreType.DMA,
            pltpu.SemaphoreType.DMA,
            pltpu.SemaphoreType.DMA,
            pltpu.SemaphoreType.DMA,
            pltpu.SemaphoreType.DMA,
        ],
        compiler_params=pltpu.CompilerParams(needs_layout_passes=False),
    )
    return run(edge_t, c_flat, log_tab)


def kernel(X, edge_idx, C):
    del X
    edge_t = jnp.transpose(edge_idx[0], (1, 0))
    out_t = _edge_field(edge_t, C.reshape(N))
    return jnp.transpose(out_t, (2, 1, 0))[None]
